# baseline (speedup 1.0000x reference)
import functools

import jax
import jax.numpy as jnp
from jax import lax
from jax.experimental import pallas as pl
from jax.experimental.pallas import tpu as pltpu
from jax.experimental.pallas import tpu_sc as plsc

_N = 10000
_NP = 10240
_E = 320000
_G = 64
_H = 64
_W = 128
_L = 4
_BN = 1000
_BE = 2000
_CS = 512
_KC = 4
_NSC = _E // _CS
_NW = 32

_f32 = jnp.float32


def _edge_pre_body(eaA, eaB, We, be, Wm, bm, o0, o1, o2, o3):
    eA = jnp.maximum(
        jnp.dot(eaA[...], We[...], preferred_element_type=_f32) + be[...],
        0.0)
    eB = jnp.maximum(
        jnp.dot(eaB[...], We[...], preferred_element_type=_f32) + be[...],
        0.0)
    outs = (o0, o1, o2, o3)
    for l in range(_L):
        outs[l][...] = jnp.concatenate([
            jnp.dot(eA, Wm[l], preferred_element_type=_f32) + bm[l],
            jnp.dot(eB, Wm[l], preferred_element_type=_f32) + bm[l],
        ], axis=1)


_EH = _E // 2
_edge_pre = pl.pallas_call(
    _edge_pre_body,
    grid=(_EH // _BN,),
    in_specs=[
        pl.BlockSpec((_BN, 16), lambda i: (i, 0)),
        pl.BlockSpec((_BN, 16), lambda i: (_EH // _BN + i, 0)),
        pl.BlockSpec((16, _H), lambda i: (0, 0)),
        pl.BlockSpec((1, _H), lambda i: (0, 0)),
        pl.BlockSpec((_L, _H, _H), lambda i: (0, 0, 0)),
        pl.BlockSpec((_L, 1, _H), lambda i: (0, 0, 0)),
    ],
    out_specs=[pl.BlockSpec((_BN, _W), lambda i: (i, 0))] * _L,
    out_shape=[jax.ShapeDtypeStruct((_EH, _W), _f32)] * _L,
)


def _node_pre_body(x, Wn, bn, Wm0, ho, hwo):
    h = jnp.maximum(
        jnp.dot(x[...], Wn[...], preferred_element_type=_f32) + bn[...], 0.0)
    ho[...] = h
    hw = jnp.dot(h, Wm0[...], preferred_element_type=_f32)
    hwo[...] = jnp.concatenate([hw, jnp.zeros((_BN, _W - _H), _f32)], axis=1)


_node_pre = pl.pallas_call(
    _node_pre_body,
    grid=(_N // _BN,),
    in_specs=[
        pl.BlockSpec((_BN, 128), lambda i: (i, 0)),
        pl.BlockSpec((128, _H), lambda i: (0, 0)),
        pl.BlockSpec((1, _H), lambda i: (0, 0)),
        pl.BlockSpec((_H, _H), lambda i: (0, 0)),
    ],
    out_specs=[
        pl.BlockSpec((_BN, _H), lambda i: (i, 0)),
        pl.BlockSpec((_BN, _W), lambda i: (i, 0)),
    ],
    out_shape=[
        jax.ShapeDtypeStruct((_N, _H), _f32),
        jax.ShapeDtypeStruct((_N, _W), _f32),
    ],
)


def _upd_core(h, p, Wut, Wubp, bu):
    q = p[0] + p[1]
    deg = q[:, _H:_H + 1]
    inv = 1.0 / jnp.maximum(deg, 1.0)
    z = (jnp.dot(h[...], Wut[...], preferred_element_type=_f32)
         + jnp.dot(q * inv, Wubp[...], preferred_element_type=_f32)
         + bu[...])
    return jnp.maximum(z, 0.0) + h[...]


def _upd_body(h, p, Wut, Wubp, bu, Wmn, ho, hwo):
    hn = _upd_core(h, p, Wut, Wubp, bu)
    ho[...] = hn
    hw = jnp.dot(hn, Wmn[...], preferred_element_type=_f32)
    hwo[...] = jnp.concatenate([hw, jnp.zeros((_BN, _W - _H), _f32)], axis=1)


def _upd_last_body(h, p, Wut, Wubp, bu, ho):
    ho[...] = _upd_core(h, p, Wut, Wubp, bu)


_upd_in_specs = [
    pl.BlockSpec((_BN, _H), lambda i: (i, 0)),
    pl.BlockSpec((2, _BN, _W), lambda i: (0, i, 0)),
    pl.BlockSpec((_H, _H), lambda i: (0, 0)),
    pl.BlockSpec((_W, _H), lambda i: (0, 0)),
    pl.BlockSpec((1, _H), lambda i: (0, 0)),
]

_upd = pl.pallas_call(
    _upd_body,
    grid=(_N // _BN,),
    in_specs=_upd_in_specs + [pl.BlockSpec((_H, _H), lambda i: (0, 0))],
    out_specs=[
        pl.BlockSpec((_BN, _H), lambda i: (i, 0)),
        pl.BlockSpec((_BN, _W), lambda i: (i, 0)),
    ],
    out_shape=[
        jax.ShapeDtypeStruct((_N, _H), _f32),
        jax.ShapeDtypeStruct((_N, _W), _f32),
    ],
)

_upd_last = pl.pallas_call(
    _upd_last_body,
    grid=(_N // _BN,),
    in_specs=_upd_in_specs,
    out_specs=pl.BlockSpec((_BN, _H), lambda i: (i, 0)),
    out_shape=jax.ShapeDtypeStruct((_N, _H), _f32),
)


def _pool_body(ne, b3, sums, cnt):
    i = pl.program_id(0)

    @pl.when(i == 0)
    def _init():
        sums[...] = jnp.zeros((_G, _H), _f32)
        cnt[...] = jnp.zeros((_G, _H), _f32)

    bm = b3[0]
    ohT = (lax.broadcasted_iota(jnp.int32, (_G, _BN), 0) == bm).astype(_f32)
    sums[...] += jnp.dot(ohT, ne[...], preferred_element_type=_f32)
    c = jnp.sum(ohT, axis=1, keepdims=True)
    cnt[...] += jnp.broadcast_to(c, (_G, _H))


_pool = pl.pallas_call(
    _pool_body,
    grid=(_N // _BN,),
    in_specs=[
        pl.BlockSpec((_BN, _H), lambda i: (i, 0)),
        pl.BlockSpec((1, 1, _BN), lambda i: (i, 0, 0)),
    ],
    out_specs=[
        pl.BlockSpec((_G, _H), lambda i: (0, 0)),
        pl.BlockSpec((_G, _H), lambda i: (0, 0)),
    ],
    out_shape=[
        jax.ShapeDtypeStruct((_G, _H), _f32),
        jax.ShapeDtypeStruct((_G, _H), _f32),
    ],
)


def _score_body(ne, bc, nc, sums, cnt, WbT, bb, ge_out, loss_out):
    i = pl.program_id(0)
    ge = sums[...] / jnp.maximum(cnt[...], 1.0)
    u = jnp.dot(ge, WbT[...], preferred_element_type=_f32)

    @pl.when(i == 0)
    def _init():
        ge_out[...] = ge
        loss_out[...] = jnp.zeros((1, 1), _f32)

    io = lax.broadcasted_iota(jnp.int32, (_BN, _G), 1)
    ohp = (io == bc[...]).astype(_f32)
    ohn = (io == nc[...]).astype(_f32)
    nev = ne[...]
    pos = jnp.sum(nev * jnp.dot(ohp, u, preferred_element_type=_f32),
                  axis=1, keepdims=True) + bb[0, 0]
    neg = jnp.sum(nev * jnp.dot(ohn, u, preferred_element_type=_f32),
                  axis=1, keepdims=True) + bb[0, 0]
    sp_pos = jnp.maximum(-pos, 0.0) + jnp.log(1.0 + jnp.exp(-jnp.abs(pos)))
    sp_neg = jnp.maximum(neg, 0.0) + jnp.log(1.0 + jnp.exp(-jnp.abs(neg)))
    tot = (jnp.sum(sp_pos) + jnp.sum(sp_neg)) / float(_N)
    loss_out[...] = loss_out[...] + tot


_score = pl.pallas_call(
    _score_body,
    grid=(_N // _BN,),
    in_specs=[
        pl.BlockSpec((_BN, _H), lambda i: (i, 0)),
        pl.BlockSpec((_BN, 1), lambda i: (i, 0)),
        pl.BlockSpec((_BN, 1), lambda i: (i, 0)),
        pl.BlockSpec((_G, _H), lambda i: (0, 0)),
        pl.BlockSpec((_G, _H), lambda i: (0, 0)),
        pl.BlockSpec((_H, _H), lambda i: (0, 0)),
        pl.BlockSpec((1, 1), lambda i: (0, 0)),
    ],
    out_specs=[
        pl.BlockSpec((_G, _H), lambda i: (0, 0)),
        pl.BlockSpec((1, 1), lambda i: (0, 0)),
    ],
    out_shape=[
        jax.ShapeDtypeStruct((_G, _H), _f32),
        jax.ShapeDtypeStruct((1, 1), _f32),
    ],
)


def _make_sc_layer():
    rows_per_sub = _NP // 16
    cs = 64
    nch = _E // cs
    npairs = (nch // _NW + 2) // 2

    scratch = (
        [pltpu.VMEM((cs,), jnp.int32)] * 8
        + [pltpu.VMEM((cs, _W), _f32)] * 2
        + [pltpu.VMEM((cs // 2, _W), _f32)] * 2
        + [pltpu.VMEM_SHARED((_NP, _W), _f32)]
        + [pltpu.SemaphoreType.DMA] * 12
    )

    @functools.partial(
        pl.kernel,
        mesh=plsc.VectorSubcoreMesh(core_axis_name="c", subcore_axis_name="s"),
        out_type=jax.ShapeDtypeStruct((2, _NP, _W), _f32),
        scratch_types=scratch,
    )
    def k(hw_hbm, ewb_hbm, src_hbm, dst_hbm, out_hbm,
          src0, src1, src2, src3, dst0, dst1, dst2, dst3,
          rows0, rows1, e0, e1, acc,
          ss0, ss1, ss2, ss3, ds0, ds1, ds2, ds3, es0, es1, gs0, gs1):
        srcb, dstb = (src0, src1, src2, src3), (dst0, dst1, dst2, dst3)
        rowsb, eb = (rows0, rows1), (e0, e1)
        ssem, dsem = (ss0, ss1, ss2, ss3), (ds0, ds1, ds2, ds3)
        esem, gsem = (es0, es1), (gs0, gs1)
        c = lax.axis_index("c")
        s = lax.axis_index("s")
        w = s * 2 + c
        io16 = lax.iota(jnp.int32, 16)
        one_first = jnp.maximum(1 - io16, 0).astype(_f32)

        def zb(i, carry):
            r = i // 8
            col = (i % 8) * 16
            rows0[r, pl.ds(col, 16)] = jnp.zeros((16,), _f32)
            return carry

        lax.fori_loop(0, cs * 8, zb, 0)
        for t in range(10):
            pltpu.sync_copy(rows0,
                            acc.at[pl.ds(s * rows_per_sub + t * cs, cs)])
        plsc.subcore_barrier()

        nt = (nch - 1 - w) // _NW + 1

        def load_idx(kk, b4):
            base = (w + kk * _NW) * cs

            @pl.when(kk < nt)
            def _():
                pltpu.async_copy(src_hbm.at[pl.ds(base, cs)], srcb[b4],
                                 ssem[b4])
                pltpu.async_copy(dst_hbm.at[pl.ds(base, cs)], dstb[b4],
                                 dsem[b4])

        def load_e(kk, b2):
            ch = w + kk * _NW

            @pl.when(kk < nt)
            def _():
                pltpu.async_copy(ewb_hbm.at[pl.ds(ch * (cs // 2), cs // 2)],
                                 eb[b2], esem[b2])

        def fire_gather(kk, b4, b2):
            base = (w + kk * _NW) * cs

            @pl.when(kk < nt)
            def _():
                pltpu.make_async_copy(src_hbm.at[pl.ds(base, cs)], srcb[b4],
                                      ssem[b4]).wait()
                pltpu.async_copy(hw_hbm.at[srcb[b4]], rowsb[b2], gsem[b2])

        def work(kk, b4, b2):
            ch = w + kk * _NW
            base = ch * cs

            @pl.when(kk < nt)
            def _():
                pltpu.make_async_copy(hw_hbm.at[srcb[b4]], rowsb[b2],
                                      gsem[b2]).wait()
                pltpu.make_async_copy(
                    ewb_hbm.at[pl.ds(ch * (cs // 2), cs // 2)], eb[b2],
                    esem[b2]).wait()

                @plsc.parallel_loop(0, cs // 2, 1, unroll=4)
                def comp(r):
                    for j in range(_H // 16):
                        a = rowsb[b2][r, pl.ds(j * 16, 16)]
                        v = eb[b2][r, pl.ds(j * 16, 16)]
                        rowsb[b2][r, pl.ds(j * 16, 16)] = (
                            jnp.maximum(a + v, 0.0))
                        a2 = rowsb[b2][cs // 2 + r, pl.ds(j * 16, 16)]
                        v2 = eb[b2][r, pl.ds(_H + j * 16, 16)]
                        rowsb[b2][cs // 2 + r, pl.ds(j * 16, 16)] = (
                            jnp.maximum(a2 + v2, 0.0))
                    rowsb[b2][r, pl.ds(_H, 16)] = one_first
                    rowsb[b2][cs // 2 + r, pl.ds(_H, 16)] = one_first
                pltpu.make_async_copy(dst_hbm.at[pl.ds(base, cs)], dstb[b4],
                                      dsem[b4]).wait()
                pltpu.sync_copy(rowsb[b2], acc.at[dstb[b4]], add=True)

        load_idx(0, 0)
        load_idx(1, 1)
        load_idx(2, 2)
        load_e(0, 0)
        load_e(1, 1)
        fire_gather(0, 0, 0)

        ngroups = (npairs * 2 + 3) // 4 + 1

        def group(i, carry):
            k = 4 * i
            for b in range(4):
                fire_gather(k + b + 1, (b + 1) % 4, (b + 1) % 2)
                work(k + b, b, b % 2)
                load_idx(k + b + 3, (b + 3) % 4)
                load_e(k + b + 2, b % 2)
            return carry

        lax.fori_loop(0, ngroups, group, 0)
        plsc.subcore_barrier()

        for t in range(10):
            off = s * rows_per_sub + t * cs
            pltpu.sync_copy(acc.at[pl.ds(off, cs)], rows0)
            pltpu.sync_copy(rows0, out_hbm.at[c, pl.ds(off, cs)])

    return k


def kernel(x, edge_index, edge_attr, batch, params):
    _sc_layer = _make_sc_layer()

    def _pack_idx(v):
        return jnp.concatenate(
            [v[:_EH].reshape(-1, 32), v[_EH:].reshape(-1, 32)],
            axis=1).reshape(-1)

    src1 = _pack_idx(edge_index[0])
    dst1 = _pack_idx(edge_index[1])
    Wm = jnp.stack(params["W_msg"])
    bm = jnp.stack(params["b_msg"]).reshape(_L, 1, _H)

    ewb = _edge_pre(edge_attr, edge_attr, params["W_edge"],
                    params["b_edge"].reshape(1, _H), Wm, bm)
    h, hw = _node_pre(x, params["W_node"], params["b_node"].reshape(1, _H),
                      params["W_msg"][0])

    for l in range(_L):
        p = _sc_layer(hw, ewb[l], src1, dst1)
        Wut = params["W_upd"][l][:_H]
        Wubp = jnp.concatenate(
            [params["W_upd"][l][_H:], jnp.zeros((_W - _H, _H), _f32)], axis=0)
        bu = params["b_upd"][l].reshape(1, _H)
        if l < _L - 1:
            h, hw = _upd(h, p, Wut, Wubp, bu, params["W_msg"][l + 1])
        else:
            h = _upd_last(h, p, Wut, Wubp, bu)

    b3 = batch.reshape(_N // _BN, 1, _BN)
    sums, cnt = _pool(h, b3)
    bc = batch.reshape(_N, 1)
    neg_off = jax.random.randint(jax.random.key(42), (_N,), 1, _G)
    nc = ((batch + neg_off) % _G).reshape(_N, 1)
    ge, loss_arr = _score(h, bc, nc, sums, cnt,
                          params["W_bil"].T,
                          params["b_bil"].reshape(1, 1))
    return loss_arr[0, 0], h, ge


# speedup vs baseline: 1.0371x; 1.0371x over previous
import functools

import jax
import jax.numpy as jnp
from jax import lax
from jax.experimental import pallas as pl
from jax.experimental.pallas import tpu as pltpu
from jax.experimental.pallas import tpu_sc as plsc

_N = 10000
_NP = 10240
_E = 320000
_G = 64
_H = 64
_W = 128
_L = 4
_BN = 1000
_BE = 2000
_CS = 512
_KC = 4
_NSC = _E // _CS
_NW = 32

_f32 = jnp.float32


def _edge_pre_body(eaA, eaB, We, be, Wm, bm, o0, o1, o2, o3):
    eA = jnp.maximum(
        jnp.dot(eaA[...], We[...], preferred_element_type=_f32) + be[...],
        0.0)
    eB = jnp.maximum(
        jnp.dot(eaB[...], We[...], preferred_element_type=_f32) + be[...],
        0.0)
    outs = (o0, o1, o2, o3)
    for l in range(_L):
        outs[l][...] = jnp.concatenate([
            jnp.dot(eA, Wm[l], preferred_element_type=_f32) + bm[l],
            jnp.dot(eB, Wm[l], preferred_element_type=_f32) + bm[l],
        ], axis=1)


_EH = _E // 2
_edge_pre = pl.pallas_call(
    _edge_pre_body,
    grid=(_EH // _BN,),
    in_specs=[
        pl.BlockSpec((_BN, 16), lambda i: (i, 0)),
        pl.BlockSpec((_BN, 16), lambda i: (_EH // _BN + i, 0)),
        pl.BlockSpec((16, _H), lambda i: (0, 0)),
        pl.BlockSpec((1, _H), lambda i: (0, 0)),
        pl.BlockSpec((_L, _H, _H), lambda i: (0, 0, 0)),
        pl.BlockSpec((_L, 1, _H), lambda i: (0, 0, 0)),
    ],
    out_specs=[pl.BlockSpec((_BN, _W), lambda i: (i, 0))] * _L,
    out_shape=[jax.ShapeDtypeStruct((_EH, _W), _f32)] * _L,
)


def _node_pre_body(x, Wn, bn, Wm0, ho, hwo):
    h = jnp.maximum(
        jnp.dot(x[...], Wn[...], preferred_element_type=_f32) + bn[...], 0.0)
    ho[...] = h
    hw = jnp.dot(h, Wm0[...], preferred_element_type=_f32)
    hwo[...] = jnp.concatenate([hw, jnp.zeros((_BN, _W - _H), _f32)], axis=1)


_node_pre = pl.pallas_call(
    _node_pre_body,
    grid=(_N // _BN,),
    in_specs=[
        pl.BlockSpec((_BN, 128), lambda i: (i, 0)),
        pl.BlockSpec((128, _H), lambda i: (0, 0)),
        pl.BlockSpec((1, _H), lambda i: (0, 0)),
        pl.BlockSpec((_H, _H), lambda i: (0, 0)),
    ],
    out_specs=[
        pl.BlockSpec((_BN, _H), lambda i: (i, 0)),
        pl.BlockSpec((_BN, _W), lambda i: (i, 0)),
    ],
    out_shape=[
        jax.ShapeDtypeStruct((_N, _H), _f32),
        jax.ShapeDtypeStruct((_N, _W), _f32),
    ],
)


def _upd_core(h, p, Wut, Wubp, bu):
    q = p[0] + p[1]
    deg = q[:, _H:_H + 1]
    inv = 1.0 / jnp.maximum(deg, 1.0)
    z = (jnp.dot(h[...], Wut[...], preferred_element_type=_f32)
         + jnp.dot(q * inv, Wubp[...], preferred_element_type=_f32)
         + bu[...])
    return jnp.maximum(z, 0.0) + h[...]


def _upd_body(h, p, Wut, Wubp, bu, Wmn, ho, hwo):
    hn = _upd_core(h, p, Wut, Wubp, bu)
    ho[...] = hn
    hw = jnp.dot(hn, Wmn[...], preferred_element_type=_f32)
    hwo[...] = jnp.concatenate([hw, jnp.zeros((_BN, _W - _H), _f32)], axis=1)


def _upd_last_body(h, p, Wut, Wubp, bu, ho):
    ho[...] = _upd_core(h, p, Wut, Wubp, bu)


_upd_in_specs = [
    pl.BlockSpec((_BN, _H), lambda i: (i, 0)),
    pl.BlockSpec((2, _BN, _W), lambda i: (0, i, 0)),
    pl.BlockSpec((_H, _H), lambda i: (0, 0)),
    pl.BlockSpec((_W, _H), lambda i: (0, 0)),
    pl.BlockSpec((1, _H), lambda i: (0, 0)),
]

_upd = pl.pallas_call(
    _upd_body,
    grid=(_N // _BN,),
    in_specs=_upd_in_specs + [pl.BlockSpec((_H, _H), lambda i: (0, 0))],
    out_specs=[
        pl.BlockSpec((_BN, _H), lambda i: (i, 0)),
        pl.BlockSpec((_BN, _W), lambda i: (i, 0)),
    ],
    out_shape=[
        jax.ShapeDtypeStruct((_N, _H), _f32),
        jax.ShapeDtypeStruct((_N, _W), _f32),
    ],
)

_upd_last = pl.pallas_call(
    _upd_last_body,
    grid=(_N // _BN,),
    in_specs=_upd_in_specs,
    out_specs=pl.BlockSpec((_BN, _H), lambda i: (i, 0)),
    out_shape=jax.ShapeDtypeStruct((_N, _H), _f32),
)


def _pool_body(ne, b3, sums, cnt):
    i = pl.program_id(0)

    @pl.when(i == 0)
    def _init():
        sums[...] = jnp.zeros((_G, _H), _f32)
        cnt[...] = jnp.zeros((_G, _H), _f32)

    bm = b3[0]
    ohT = (lax.broadcasted_iota(jnp.int32, (_G, _BN), 0) == bm).astype(_f32)
    sums[...] += jnp.dot(ohT, ne[...], preferred_element_type=_f32)
    c = jnp.sum(ohT, axis=1, keepdims=True)
    cnt[...] += jnp.broadcast_to(c, (_G, _H))


_pool = pl.pallas_call(
    _pool_body,
    grid=(_N // _BN,),
    in_specs=[
        pl.BlockSpec((_BN, _H), lambda i: (i, 0)),
        pl.BlockSpec((1, 1, _BN), lambda i: (i, 0, 0)),
    ],
    out_specs=[
        pl.BlockSpec((_G, _H), lambda i: (0, 0)),
        pl.BlockSpec((_G, _H), lambda i: (0, 0)),
    ],
    out_shape=[
        jax.ShapeDtypeStruct((_G, _H), _f32),
        jax.ShapeDtypeStruct((_G, _H), _f32),
    ],
)


def _score_body(ne, bc, nc, sums, cnt, WbT, bb, ge_out, loss_out):
    i = pl.program_id(0)
    ge = sums[...] / jnp.maximum(cnt[...], 1.0)
    u = jnp.dot(ge, WbT[...], preferred_element_type=_f32)

    @pl.when(i == 0)
    def _init():
        ge_out[...] = ge
        loss_out[...] = jnp.zeros((1, 1), _f32)

    io = lax.broadcasted_iota(jnp.int32, (_BN, _G), 1)
    ohp = (io == bc[...]).astype(_f32)
    ohn = (io == nc[...]).astype(_f32)
    nev = ne[...]
    pos = jnp.sum(nev * jnp.dot(ohp, u, preferred_element_type=_f32),
                  axis=1, keepdims=True) + bb[0, 0]
    neg = jnp.sum(nev * jnp.dot(ohn, u, preferred_element_type=_f32),
                  axis=1, keepdims=True) + bb[0, 0]
    sp_pos = jnp.maximum(-pos, 0.0) + jnp.log(1.0 + jnp.exp(-jnp.abs(pos)))
    sp_neg = jnp.maximum(neg, 0.0) + jnp.log(1.0 + jnp.exp(-jnp.abs(neg)))
    tot = (jnp.sum(sp_pos) + jnp.sum(sp_neg)) / float(_N)
    loss_out[...] = loss_out[...] + tot


_score = pl.pallas_call(
    _score_body,
    grid=(_N // _BN,),
    in_specs=[
        pl.BlockSpec((_BN, _H), lambda i: (i, 0)),
        pl.BlockSpec((_BN, 1), lambda i: (i, 0)),
        pl.BlockSpec((_BN, 1), lambda i: (i, 0)),
        pl.BlockSpec((_G, _H), lambda i: (0, 0)),
        pl.BlockSpec((_G, _H), lambda i: (0, 0)),
        pl.BlockSpec((_H, _H), lambda i: (0, 0)),
        pl.BlockSpec((1, 1), lambda i: (0, 0)),
    ],
    out_specs=[
        pl.BlockSpec((_G, _H), lambda i: (0, 0)),
        pl.BlockSpec((1, 1), lambda i: (0, 0)),
    ],
    out_shape=[
        jax.ShapeDtypeStruct((_G, _H), _f32),
        jax.ShapeDtypeStruct((1, 1), _f32),
    ],
)


def _make_sc_layer():
    rows_per_sub = _NP // 16
    cs = 64
    nch = _E // cs
    npairs = (nch // _NW + 2) // 2

    scratch = (
        [pltpu.VMEM((cs,), jnp.int32)] * 16
        + [pltpu.VMEM((cs, _W), _f32)] * 4
        + [pltpu.VMEM((cs // 2, _W), _f32)] * 2
        + [pltpu.VMEM_SHARED((_NP, _W), _f32)]
        + [pltpu.SemaphoreType.DMA] * 26
    )

    @functools.partial(
        pl.kernel,
        mesh=plsc.VectorSubcoreMesh(core_axis_name="c", subcore_axis_name="s"),
        out_type=jax.ShapeDtypeStruct((2, _NP, _W), _f32),
        scratch_types=scratch,
    )
    def k(hw_hbm, ewb_hbm, src_hbm, dst_hbm, out_hbm, *bufs):
        srcb, dstb = bufs[0:8], bufs[8:16]
        rowsb, eb = bufs[16:20], bufs[20:22]
        acc = bufs[22]
        ssem, dsem = bufs[23:31], bufs[31:39]
        esem, gsem, scsem = bufs[39:41], bufs[41:45], bufs[45:49]
        c = lax.axis_index("c")
        s = lax.axis_index("s")
        w = s * 2 + c
        io16 = lax.iota(jnp.int32, 16)
        one_first = jnp.maximum(1 - io16, 0).astype(_f32)

        def zb(i, carry):
            r = i // 8
            col = (i % 8) * 16
            rowsb[0][r, pl.ds(col, 16)] = jnp.zeros((16,), _f32)
            return carry

        lax.fori_loop(0, cs * 8, zb, 0)
        for t in range(10):
            pltpu.sync_copy(rowsb[0],
                            acc.at[pl.ds(s * rows_per_sub + t * cs, cs)])
        plsc.subcore_barrier()

        nt = (nch - 1 - w) // _NW + 1

        def load_idx(kk, b8):
            base = (w + kk * _NW) * cs

            @pl.when(kk < nt)
            def _():
                pltpu.async_copy(src_hbm.at[pl.ds(base, cs)], srcb[b8],
                                 ssem[b8])
                pltpu.async_copy(dst_hbm.at[pl.ds(base, cs)], dstb[b8],
                                 dsem[b8])

        def load_e(kk, b2):
            ch = w + kk * _NW

            @pl.when(kk < nt)
            def _():
                pltpu.async_copy(ewb_hbm.at[pl.ds(ch * (cs // 2), cs // 2)],
                                 eb[b2], esem[b2])

        def fire_gather(kk, b8, b4, first):
            base = (w + kk * _NW) * cs

            @pl.when(kk < nt)
            def _():
                if not first:
                    pltpu.make_async_copy(rowsb[b4],
                                          acc.at[dstb[(b8 + 4) % 8]],
                                          scsem[b4]).wait()
                pltpu.make_async_copy(src_hbm.at[pl.ds(base, cs)], srcb[b8],
                                      ssem[b8]).wait()
                pltpu.async_copy(hw_hbm.at[srcb[b8]], rowsb[b4], gsem[b4])

        def work(kk, b8, b4, b2):
            ch = w + kk * _NW
            base = ch * cs

            @pl.when(kk < nt)
            def _():
                pltpu.make_async_copy(hw_hbm.at[srcb[b8]], rowsb[b4],
                                      gsem[b4]).wait()
                pltpu.make_async_copy(
                    ewb_hbm.at[pl.ds(ch * (cs // 2), cs // 2)], eb[b2],
                    esem[b2]).wait()

                def comp(r, cc):
                    for j in range(_H // 16):
                        a = rowsb[b4][r, pl.ds(j * 16, 16)]
                        v = eb[b2][r, pl.ds(j * 16, 16)]
                        rowsb[b4][r, pl.ds(j * 16, 16)] = (
                            jnp.maximum(a + v, 0.0))
                        a2 = rowsb[b4][cs // 2 + r, pl.ds(j * 16, 16)]
                        v2 = eb[b2][r, pl.ds(_H + j * 16, 16)]
                        rowsb[b4][cs // 2 + r, pl.ds(j * 16, 16)] = (
                            jnp.maximum(a2 + v2, 0.0))
                    rowsb[b4][r, pl.ds(_H, 16)] = one_first
                    rowsb[b4][cs // 2 + r, pl.ds(_H, 16)] = one_first
                    return cc

                lax.fori_loop(0, cs // 2, comp, 0)
                pltpu.make_async_copy(dst_hbm.at[pl.ds(base, cs)], dstb[b8],
                                      dsem[b8]).wait()
                pltpu.async_copy(rowsb[b4], acc.at[dstb[b8]], scsem[b4],
                                 add=True)

        def slot(kk, b, first_pass):
            fire_gather(kk + 1, (b + 1) % 8, (b + 1) % 4,
                        first_pass and (b + 1 < 4))
            work(kk, b % 8, b % 4, b % 2)
            load_idx(kk + 3, (b + 3) % 8)
            load_e(kk + 2, b % 2)

        load_idx(0, 0)
        load_idx(1, 1)
        load_idx(2, 2)
        load_e(0, 0)
        load_e(1, 1)
        fire_gather(0, 0, 0, True)

        for b in range(8):
            slot(b, b, True)

        ngroups = (nch // _NW + 8) // 8

        def group(i, carry):
            k = 8 * (i + 1)
            for b in range(8):
                slot(k + b, b, False)
            return carry

        lax.fori_loop(0, ngroups - 1, group, 0)

        for b4 in range(4):
            pltpu.make_async_copy(rowsb[b4], acc.at[dstb[b4]],
                                  scsem[b4]).wait()
        plsc.subcore_barrier()

        for t in range(10):
            off = s * rows_per_sub + t * cs
            pltpu.sync_copy(acc.at[pl.ds(off, cs)], rowsb[0])
            pltpu.sync_copy(rowsb[0], out_hbm.at[c, pl.ds(off, cs)])

    return k


def kernel(x, edge_index, edge_attr, batch, params):
    _sc_layer = _make_sc_layer()

    def _pack_idx(v):
        return jnp.concatenate(
            [v[:_EH].reshape(-1, 32), v[_EH:].reshape(-1, 32)],
            axis=1).reshape(-1)

    src1 = _pack_idx(edge_index[0])
    dst1 = _pack_idx(edge_index[1])
    Wm = jnp.stack(params["W_msg"])
    bm = jnp.stack(params["b_msg"]).reshape(_L, 1, _H)

    ewb = _edge_pre(edge_attr, edge_attr, params["W_edge"],
                    params["b_edge"].reshape(1, _H), Wm, bm)
    h, hw = _node_pre(x, params["W_node"], params["b_node"].reshape(1, _H),
                      params["W_msg"][0])

    for l in range(_L):
        p = _sc_layer(hw, ewb[l], src1, dst1)
        Wut = params["W_upd"][l][:_H]
        Wubp = jnp.concatenate(
            [params["W_upd"][l][_H:], jnp.zeros((_W - _H, _H), _f32)], axis=0)
        bu = params["b_upd"][l].reshape(1, _H)
        if l < _L - 1:
            h, hw = _upd(h, p, Wut, Wubp, bu, params["W_msg"][l + 1])
        else:
            h = _upd_last(h, p, Wut, Wubp, bu)

    b3 = batch.reshape(_N // _BN, 1, _BN)
    sums, cnt = _pool(h, b3)
    bc = batch.reshape(_N, 1)
    neg_off = jax.random.randint(jax.random.key(42), (_N,), 1, _G)
    nc = ((batch + neg_off) % _G).reshape(_N, 1)
    ge, loss_arr = _score(h, bc, nc, sums, cnt,
                          params["W_bil"].T,
                          params["b_bil"].reshape(1, 1))
    return loss_arr[0, 0], h, ge
